# Initial kernel scaffold; baseline (speedup 1.0000x reference)
import jax
import jax.numpy as jnp
from jax.experimental import pallas as pl


def kernel(x_latent, embed_weight):
    raise NotImplementedError("write your pallas kernel here")


# speedup vs baseline: 1.6627x; 1.6627x over previous
import functools

import jax
import jax.numpy as jnp
from jax.experimental import pallas as pl

_NUM_EMBEDS = 1024
_EMBED_DIM = 256
_BETA = 0.25


def _vq_body(x_ref, e_ref, q_ref, loss_ref):
    b = pl.program_id(0)
    x = x_ref[0]
    emb = e_ref[...]
    e_sq = jnp.sum(emb * emb, axis=1, keepdims=True)
    x_sq = jnp.sum(x * x, axis=0, keepdims=True)
    cross_t = jax.lax.dot_general(
        emb.astype(jnp.bfloat16), x.astype(jnp.bfloat16),
        (((1,), (0,)), ((), ())),
        preferred_element_type=jnp.float32)
    dist = (x_sq + e_sq) - 2.0 * cross_t
    minval = jnp.min(dist, axis=0)
    iota_e = jax.lax.broadcasted_iota(
        jnp.int32, (_NUM_EMBEDS, dist.shape[1]), 0)
    ind = jnp.min(
        jnp.where(dist == minval[None, :], iota_e, _NUM_EMBEDS),
        axis=0)
    onehot_t = (iota_e == ind[None, :]).astype(jnp.float32)
    q_t = jax.lax.dot_general(
        emb, onehot_t, (((0,), (0,)), ((), ())),
        preferred_element_type=jnp.float32)
    q_ref[0] = q_t
    partial = jnp.sum(minval).reshape(1, 1)

    @pl.when(b == 0)
    def _init():
        loss_ref[...] = jnp.zeros((1, 1), jnp.float32)

    loss_ref[...] += partial


@functools.partial(jax.jit, static_argnames=())
def kernel(x_latent, embed_weight):
    B, C, H, W = x_latent.shape
    N = H * W
    x3 = x_latent.reshape(B, C, N)
    q3, loss_sum = pl.pallas_call(
        _vq_body,
        grid=(B,),
        in_specs=[
            pl.BlockSpec((1, C, N), lambda b: (b, 0, 0)),
            pl.BlockSpec((_NUM_EMBEDS, _EMBED_DIM), lambda b: (0, 0)),
        ],
        out_specs=[
            pl.BlockSpec((1, C, N), lambda b: (b, 0, 0)),
            pl.BlockSpec((1, 1), lambda b: (0, 0)),
        ],
        out_shape=[
            jax.ShapeDtypeStruct((B, C, N), jnp.float32),
            jax.ShapeDtypeStruct((1, 1), jnp.float32),
        ],
    )(x3, embed_weight)
    vq_loss = (1.0 + _BETA) * loss_sum[0, 0] / (B * C * H * W)
    return q3.reshape(B, C, H, W), vq_loss
